# baseline (speedup 1.0000x reference)
import functools

import jax
import jax.numpy as jnp
from jax import lax
from jax.experimental import pallas as pl
from jax.experimental.pallas import tpu as pltpu
from jax.experimental.pallas import tpu_sc as plsc

N_NODES = 10000
N_PAD = 10240
E_TOT = 330000
NC, NS = 2, 16
NW = NC * NS
CHUNK = 128
NBUF = 2
HBM_TILES = 5
C_A = 82
C_B = 82
CPT_MAX = max(C_A, C_B)
E_PAD = NS * (C_A + C_B) * CHUNK
ROWS_PT = N_PAD // NS
HID = 64
W3PAD = 16

@functools.cache
def _mesh():
    return plsc.VectorSubcoreMesh(core_axis_name="c", subcore_axis_name="s",
                                  num_cores=NC, num_subcores=NS)


def _seg_body(width, g_hbm, src_hbm, dst_hbm, zrows_hbm, out_hbm,
              src_v, dst_v, rows_v, acc_sh, g_sh, *sems):
    core = lax.axis_index("c")
    sid = lax.axis_index("s")
    wid = sid * NC + core
    cpt = lax.select(core == 0, C_A, C_B)
    gsems = sems
    pltpu.sync_copy(zrows_hbm, acc_sh.at[pl.ds(sid * ROWS_PT, ROWS_PT)])
    pltpu.sync_copy(g_hbm.at[pl.ds(sid * ROWS_PT, ROWS_PT)],
                    g_sh.at[pl.ds(sid * ROWS_PT, ROWS_PT)])
    pltpu.sync_copy(src_hbm.at[wid], src_v)
    pltpu.sync_copy(dst_hbm.at[wid], dst_v)
    plsc.subcore_barrier()

    def _pipe(tab):
        def _gather(j, b):
            pltpu.async_copy(tab.at[src_v.at[j]], rows_v.at[b], gsems[b])

        def _gather_wait(j, b):
            pltpu.make_async_copy(tab.at[src_v.at[j]], rows_v.at[b],
                                  gsems[b]).wait()

        for b in range(NBUF):
            _gather(b, b)

        def body(g, _):
            j0 = g * NBUF
            for b in range(NBUF):
                j = j0 + b
                _gather_wait(j, b)
                pltpu.sync_copy(rows_v.at[b], acc_sh.at[dst_v.at[j]],
                                add=True)
                nj = j + NBUF

                @pl.when(nj < cpt)
                def _():
                    _gather(nj, b)
            return ()

        lax.fori_loop(0, cpt // NBUF, body, ())

    @pl.when(sid < HBM_TILES)
    def _():
        _pipe(g_hbm)

    @pl.when(sid >= HBM_TILES)
    def _():
        _pipe(g_sh)
    plsc.subcore_barrier()
    pltpu.sync_copy(acc_sh.at[pl.ds(sid * ROWS_PT, ROWS_PT)],
                    out_hbm.at[core, pl.ds(sid * ROWS_PT, ROWS_PT)])


def _seg_sum(g, src2, dst2, zrows, width):
    return pl.kernel(
        functools.partial(_seg_body, width),
        out_type=jax.ShapeDtypeStruct((NC, N_PAD, width), jnp.float32),
        mesh=_mesh(),
        scratch_types=[
            pltpu.VMEM((CPT_MAX, CHUNK), jnp.int32),
            pltpu.VMEM((CPT_MAX, CHUNK), jnp.int32),
            pltpu.VMEM((NBUF, CHUNK, width), jnp.float32),
            pltpu.VMEM_SHARED((N_PAD, width), jnp.float32),
            pltpu.VMEM_SHARED((N_PAD, width), jnp.float32),
        ] + [pltpu.SemaphoreType.DMA] * NBUF,
        compiler_params=pltpu.CompilerParams(use_tc_tiling_on_sc=False),
    )(g, src2, dst2, zrows)


def _deg_body(ones_hbm, dst_hbm, zrows_hbm, out_hbm, ones_v, dst_v, acc_sh, sem):
    core = lax.axis_index("c")
    sid = lax.axis_index("s")
    wid = sid * NC + core
    cpt = lax.select(core == 0, C_A, C_B)
    pltpu.sync_copy(zrows_hbm, acc_sh.at[pl.ds(sid * ROWS_PT, ROWS_PT)])
    pltpu.sync_copy(ones_hbm, ones_v)
    pltpu.sync_copy(dst_hbm.at[wid], dst_v)
    plsc.subcore_barrier()

    def body(j, _):
        pltpu.async_copy(ones_v, acc_sh.at[dst_v.at[j]], sem, add=True)
        return ()

    lax.fori_loop(0, cpt, body, ())

    def drain(j, _):
        pltpu.make_async_copy(ones_v, acc_sh.at[dst_v.at[j]], sem).wait()
        return ()

    lax.fori_loop(0, cpt, drain, ())
    plsc.subcore_barrier()
    pltpu.sync_copy(acc_sh.at[pl.ds(sid * ROWS_PT, ROWS_PT)],
                    out_hbm.at[core, pl.ds(sid * ROWS_PT, ROWS_PT)])


def _deg_count(ones, dst2, zrows):
    return pl.kernel(
        _deg_body,
        out_type=jax.ShapeDtypeStruct((NC, N_PAD, W3PAD), jnp.float32),
        mesh=_mesh(),
        scratch_types=[
            pltpu.VMEM((CHUNK, W3PAD), jnp.float32),
            pltpu.VMEM((CPT_MAX, CHUNK), jnp.int32),
            pltpu.VMEM_SHARED((N_PAD, W3PAD), jnp.float32),
            pltpu.SemaphoreType.DMA,
        ],
        compiler_params=pltpu.CompilerParams(use_tc_tiling_on_sc=False),
    )(ones, dst2, zrows)


def _dis(degp_ref):
    deg = degp_ref[0, :, 0:1] + degp_ref[1, :, 0:1]
    return jnp.where(deg > 0.0, lax.rsqrt(deg), 0.0)


def _tc_first_body(x_ref, w_ref, degp_ref, o_ref):
    dis = _dis(degp_ref)
    o_ref[...] = jnp.dot(x_ref[...], w_ref[...],
                         preferred_element_type=jnp.float32) * dis


def _tc_mid_body(sp_ref, degp_ref, w_ref, b_ref, o_ref):
    dis = _dis(degp_ref)
    h = (sp_ref[0] + sp_ref[1]) * dis + b_ref[...]
    h = jnp.where(h >= 0.0, h, 0.01 * h)
    o_ref[...] = jnp.dot(h, w_ref[...],
                         preferred_element_type=jnp.float32) * dis


def _tc_fin_body(sp_ref, degp_ref, b_ref, o_ref):
    dis = _dis(degp_ref)
    o_ref[...] = (sp_ref[0, :, 0:1] + sp_ref[1, :, 0:1]) * dis + b_ref[...]


def _tc_call(body, out_shape, *args):
    return pl.pallas_call(
        body,
        out_shape=out_shape,
        in_specs=[pl.BlockSpec(memory_space=pltpu.VMEM) for _ in args],
        out_specs=pl.BlockSpec(memory_space=pltpu.VMEM),
    )(*args)


def kernel(x, edge_index, W1, b1, W2, b2, W3, b3):
    f32 = jnp.float32
    ei = edge_index.astype(jnp.int32)
    loop = jnp.arange(N_NODES, dtype=jnp.int32)
    pad = jnp.full((E_PAD - E_TOT,), N_NODES, jnp.int32)

    def _layout(flat):
        na = NS * C_A * CHUNK
        a = flat[:na].reshape(NS, C_A, CHUNK)
        a = jnp.pad(a, ((0, 0), (0, CPT_MAX - C_A), (0, 0)))
        b = flat[na:].reshape(NS, C_B, CHUNK)
        b = jnp.pad(b, ((0, 0), (0, CPT_MAX - C_B), (0, 0)))
        return jnp.stack([a, b], axis=1).reshape(NW, CPT_MAX, CHUNK)

    src2 = _layout(jnp.concatenate([ei[0], loop, pad]))
    dst2 = _layout(jnp.concatenate([ei[1], loop, pad]))

    x_pad = jnp.zeros((N_PAD, x.shape[1]), f32).at[:N_NODES].set(x)
    zrows_h = jnp.zeros((ROWS_PT, HID), f32)
    zrows_s = jnp.zeros((ROWS_PT, W3PAD), f32)
    ones = jnp.zeros((CHUNK, W3PAD), f32).at[:, 0].set(1.0)
    W3p = jnp.zeros((HID, W3PAD), f32).at[:, 0:1].set(W3)

    degp = _deg_count(ones, dst2, zrows_s)

    g1 = _tc_call(_tc_first_body,
                  jax.ShapeDtypeStruct((N_PAD, HID), f32),
                  x_pad, W1, degp)
    s1 = _seg_sum(g1, src2, dst2, zrows_h, HID)

    g2 = _tc_call(_tc_mid_body,
                  jax.ShapeDtypeStruct((N_PAD, HID), f32),
                  s1, degp, W2, b1.reshape(1, HID))
    s2 = _seg_sum(g2, src2, dst2, zrows_h, HID)

    g3 = _tc_call(_tc_mid_body,
                  jax.ShapeDtypeStruct((N_PAD, W3PAD), f32),
                  s2, degp, W3p, b2.reshape(1, HID))
    s3 = _seg_sum(g3, src2, dst2, zrows_s, W3PAD)

    out = _tc_call(_tc_fin_body,
                   jax.ShapeDtypeStruct((N_PAD, 1), f32),
                   s3, degp, b3.reshape(1, 1))
    return out[:N_NODES]


# speedup vs baseline: 1.1059x; 1.1059x over previous
import functools

import jax
import jax.numpy as jnp
from jax import lax
from jax.experimental import pallas as pl
from jax.experimental.pallas import tpu as pltpu
from jax.experimental.pallas import tpu_sc as plsc

N_NODES = 10000
N_PAD = 10240
E_TOT = 330000
NC, NS = 2, 16
NW = NC * NS
CHUNK = 128
NBUF = 2
C_A = 82
C_B = 82
CPT_MAX = max(C_A, C_B)
E_PAD = NS * (C_A + C_B) * CHUNK
ROWS_PT = N_PAD // NS
HID = 64
W3PAD = 16

@functools.cache
def _mesh():
    return plsc.VectorSubcoreMesh(core_axis_name="c", subcore_axis_name="s",
                                  num_cores=NC, num_subcores=NS)


def _seg_body(width, g_hbm, src_hbm, dst_hbm, zrows_hbm, out_hbm,
              src_v, dst_v, rows_v, acc_sh, g_sh, *sems):
    core = lax.axis_index("c")
    sid = lax.axis_index("s")
    wid = sid * NC + core
    cpt = lax.select(core == 0, C_A, C_B)
    gsems = sems
    pltpu.sync_copy(zrows_hbm, acc_sh.at[pl.ds(sid * ROWS_PT, ROWS_PT)])
    pltpu.sync_copy(g_hbm.at[pl.ds(sid * ROWS_PT, ROWS_PT)],
                    g_sh.at[pl.ds(sid * ROWS_PT, ROWS_PT)])
    pltpu.sync_copy(src_hbm.at[wid], src_v)
    pltpu.sync_copy(dst_hbm.at[wid], dst_v)
    plsc.subcore_barrier()

    def _gather(j, b):
        pltpu.async_copy(g_sh.at[src_v.at[j]], rows_v.at[b], gsems[b])

    def _gather_wait(j, b):
        pltpu.make_async_copy(g_sh.at[src_v.at[j]], rows_v.at[b],
                              gsems[b]).wait()

    for b in range(NBUF):
        _gather(b, b)

    def body(g, _):
        j0 = g * NBUF
        for b in range(NBUF):
            j = j0 + b
            _gather_wait(j, b)
            pltpu.sync_copy(rows_v.at[b], acc_sh.at[dst_v.at[j]], add=True)
            nj = j + NBUF

            @pl.when(nj < cpt)
            def _():
                _gather(nj, b)
        return ()

    lax.fori_loop(0, cpt // NBUF, body, ())
    plsc.subcore_barrier()
    pltpu.sync_copy(acc_sh.at[pl.ds(sid * ROWS_PT, ROWS_PT)],
                    out_hbm.at[core, pl.ds(sid * ROWS_PT, ROWS_PT)])


def _seg_sum(g, src2, dst2, zrows, width):
    return pl.kernel(
        functools.partial(_seg_body, width),
        out_type=jax.ShapeDtypeStruct((NC, N_PAD, width), jnp.float32),
        mesh=_mesh(),
        scratch_types=[
            pltpu.VMEM((CPT_MAX, CHUNK), jnp.int32),
            pltpu.VMEM((CPT_MAX, CHUNK), jnp.int32),
            pltpu.VMEM((NBUF, CHUNK, width), jnp.float32),
            pltpu.VMEM_SHARED((N_PAD, width), jnp.float32),
            pltpu.VMEM_SHARED((N_PAD, width), jnp.float32),
        ] + [pltpu.SemaphoreType.DMA] * NBUF,
        compiler_params=pltpu.CompilerParams(use_tc_tiling_on_sc=False),
    )(g, src2, dst2, zrows)


def _deg_body(ones_hbm, dst_hbm, zrows_hbm, out_hbm, ones_v, dst_v, acc_sh, sem):
    core = lax.axis_index("c")
    sid = lax.axis_index("s")
    wid = sid * NC + core
    cpt = lax.select(core == 0, C_A, C_B)
    pltpu.sync_copy(zrows_hbm, acc_sh.at[pl.ds(sid * ROWS_PT, ROWS_PT)])
    pltpu.sync_copy(ones_hbm, ones_v)
    pltpu.sync_copy(dst_hbm.at[wid], dst_v)
    plsc.subcore_barrier()

    def body(j, _):
        pltpu.async_copy(ones_v, acc_sh.at[dst_v.at[j]], sem, add=True)
        return ()

    lax.fori_loop(0, cpt, body, ())

    def drain(j, _):
        pltpu.make_async_copy(ones_v, acc_sh.at[dst_v.at[j]], sem).wait()
        return ()

    lax.fori_loop(0, cpt, drain, ())
    plsc.subcore_barrier()
    pltpu.sync_copy(acc_sh.at[pl.ds(sid * ROWS_PT, ROWS_PT)],
                    out_hbm.at[core, pl.ds(sid * ROWS_PT, ROWS_PT)])


def _deg_count(ones, dst2, zrows):
    return pl.kernel(
        _deg_body,
        out_type=jax.ShapeDtypeStruct((NC, N_PAD, W3PAD), jnp.float32),
        mesh=_mesh(),
        scratch_types=[
            pltpu.VMEM((CHUNK, W3PAD), jnp.float32),
            pltpu.VMEM((CPT_MAX, CHUNK), jnp.int32),
            pltpu.VMEM_SHARED((N_PAD, W3PAD), jnp.float32),
            pltpu.SemaphoreType.DMA,
        ],
        compiler_params=pltpu.CompilerParams(use_tc_tiling_on_sc=False),
    )(ones, dst2, zrows)


def _dis(degp_ref):
    deg = degp_ref[0, :, 0:1] + degp_ref[1, :, 0:1]
    return jnp.where(deg > 0.0, lax.rsqrt(deg), 0.0)


def _tc_first_body(x_ref, w_ref, degp_ref, o_ref):
    dis = _dis(degp_ref)
    o_ref[...] = jnp.dot(x_ref[...], w_ref[...],
                         preferred_element_type=jnp.float32) * dis


def _tc_mid_body(sp_ref, degp_ref, w_ref, b_ref, o_ref):
    dis = _dis(degp_ref)
    h = (sp_ref[0] + sp_ref[1]) * dis + b_ref[...]
    h = jnp.where(h >= 0.0, h, 0.01 * h)
    o_ref[...] = jnp.dot(h, w_ref[...],
                         preferred_element_type=jnp.float32) * dis


def _tc_fin_body(sp_ref, degp_ref, b_ref, o_ref):
    dis = _dis(degp_ref)
    o_ref[...] = (sp_ref[0, :, 0:1] + sp_ref[1, :, 0:1]) * dis + b_ref[...]


def _tc_call(body, out_shape, *args):
    return pl.pallas_call(
        body,
        out_shape=out_shape,
        in_specs=[pl.BlockSpec(memory_space=pltpu.VMEM) for _ in args],
        out_specs=pl.BlockSpec(memory_space=pltpu.VMEM),
    )(*args)


def kernel(x, edge_index, W1, b1, W2, b2, W3, b3):
    f32 = jnp.float32
    ei = edge_index.astype(jnp.int32)
    loop = jnp.arange(N_NODES, dtype=jnp.int32)
    pad = jnp.full((E_PAD - E_TOT,), N_NODES, jnp.int32)

    def _layout(flat):
        na = NS * C_A * CHUNK
        a = flat[:na].reshape(NS, C_A, CHUNK)
        a = jnp.pad(a, ((0, 0), (0, CPT_MAX - C_A), (0, 0)))
        b = flat[na:].reshape(NS, C_B, CHUNK)
        b = jnp.pad(b, ((0, 0), (0, CPT_MAX - C_B), (0, 0)))
        return jnp.stack([a, b], axis=1).reshape(NW, CPT_MAX, CHUNK)

    src2 = _layout(jnp.concatenate([ei[0], loop, pad]))
    dst2 = _layout(jnp.concatenate([ei[1], loop, pad]))

    x_pad = jnp.zeros((N_PAD, x.shape[1]), f32).at[:N_NODES].set(x)
    zrows_h = jnp.zeros((ROWS_PT, HID), f32)
    zrows_s = jnp.zeros((ROWS_PT, W3PAD), f32)
    ones = jnp.zeros((CHUNK, W3PAD), f32).at[:, 0].set(1.0)
    W3p = jnp.zeros((HID, W3PAD), f32).at[:, 0:1].set(W3)

    degp = _deg_count(ones, dst2, zrows_s)

    g1 = _tc_call(_tc_first_body,
                  jax.ShapeDtypeStruct((N_PAD, HID), f32),
                  x_pad, W1, degp)
    s1 = _seg_sum(g1, src2, dst2, zrows_h, HID)

    g2 = _tc_call(_tc_mid_body,
                  jax.ShapeDtypeStruct((N_PAD, HID), f32),
                  s1, degp, W2, b1.reshape(1, HID))
    s2 = _seg_sum(g2, src2, dst2, zrows_h, HID)

    g3 = _tc_call(_tc_mid_body,
                  jax.ShapeDtypeStruct((N_PAD, W3PAD), f32),
                  s2, degp, W3p, b2.reshape(1, HID))
    s3 = _seg_sum(g3, src2, dst2, zrows_s, W3PAD)

    out = _tc_call(_tc_fin_body,
                   jax.ShapeDtypeStruct((N_PAD, 1), f32),
                   s3, degp, b3.reshape(1, 1))
    return out[:N_NODES]
